# baseline (speedup 1.0000x reference)
import functools

import numpy as np
import jax
import jax.numpy as jnp
from jax import lax
from jax.experimental import pallas as pl
from jax.experimental.pallas import tpu as pltpu
from jax.experimental.pallas import tpu_sc as plsc

NBOX = 1000
L = 16
NPAD = 1024
NCHUNK = NPAD // L
NB = 4
NC = 21
NCLS = NC - 1
NPAIR = NB * NCLS
NWORK = 32
TH_CONF = 0.05
TH_IOU = 0.5
NEG = float("-inf")
BIGI = np.int32(2**30)
PLEN = NPAD + 2 * L
DUMP = PLEN - 1
KCOMP = 24


def _body(conf_hbm, loc_hbm, anch_hbm, out_hbm,
          conf_v, cls_v, loc_v, anch_v,
          cx1_v, cy1_v, cx2_v, cy2_v, car_v, cs_v, ss_v, cidx_v, si_v,
          o0_v, o1_v, o2_v, o3_v, o4_v):
    cid = lax.axis_index("c")
    sid = lax.axis_index("s")
    wid = sid * 2 + cid
    lane = lax.iota(jnp.int32, L)

    pltpu.sync_copy(anch_hbm, anch_v)

    negv = jnp.full((L,), NEG, jnp.float32)
    bigv = jnp.full((L,), BIGI, jnp.int32)
    zi = jnp.zeros((L,), jnp.int32)

    def arg_update(sv, iv, posv, best_s, best_i, best_p):
        c2 = (sv > best_s) | ((sv == best_s) & (iv < best_i))
        return (jnp.where(c2, sv, best_s),
                jnp.where(c2, iv, best_i),
                jnp.where(c2, posv, best_p))

    def finalize(best_s, best_i, best_p):
        mval = jnp.max(best_s)
        cand = jnp.where(best_s == mval, best_i, BIGI)
        bidx = jnp.min(cand)
        bpos = jnp.min(jnp.where(cand == bidx, best_p, BIGI))
        return (mval, bidx, bpos)

    def do_pair(pair):
        b = pair // NCLS
        cls = pair % NCLS + 1
        pltpu.sync_copy(conf_hbm.at[b], conf_v)
        pltpu.sync_copy(conf_hbm.at[b, pl.ds(cls, 1)], cls_v)
        pltpu.sync_copy(loc_hbm.at[b], loc_v)

        def chunk_body(i, cnt):
            sl = pl.ds(i * L, L)
            m = conf_v[0, sl]
            for c in range(1, NC):
                m = jnp.maximum(m, conf_v[c, sl])
            z = jnp.exp(conf_v[0, sl] - m)
            for c in range(1, NC):
                z = z + jnp.exp(conf_v[c, sl] - m)
            s = jnp.exp(cls_v[0, sl] - m) / z

            a0 = anch_v[0, sl]
            a1 = anch_v[1, sl]
            a2 = anch_v[2, sl]
            a3 = anch_v[3, sl]
            cx = a0 + loc_v[0, sl] * 0.1 * a2
            cy = a1 + loc_v[1, sl] * 0.1 * a3
            w = a2 * jnp.exp(loc_v[2, sl] * 0.2)
            h = a3 * jnp.exp(loc_v[3, sl] * 0.2)
            x1 = cx - w / 2.0
            y1 = cy - h / 2.0
            x2 = cx + w / 2.0
            y2 = cy + h / 2.0
            area = (x2 - x1) * (y2 - y1)

            mask = s >= TH_CONF
            csum = lax.cumsum(mask.astype(jnp.int32))
            pos = jnp.where(mask, cnt + csum - 1, DUMP)
            plsc.store_scatter(cx1_v, [pos], x1)
            plsc.store_scatter(cy1_v, [pos], y1)
            plsc.store_scatter(cx2_v, [pos], x2)
            plsc.store_scatter(cy2_v, [pos], y2)
            plsc.store_scatter(car_v, [pos], area)
            plsc.store_scatter(cs_v, [pos], s)
            plsc.store_scatter(cidx_v, [pos], lane + i * L)

            zv = jnp.zeros((L,), jnp.float32)
            for o in (o0_v, o1_v, o2_v, o3_v, o4_v):
                o[0, sl] = zv

            return cnt + jnp.max(csum)

        cnt = lax.fori_loop(0, NCHUNK, chunk_body, np.int32(0))

        def pad_live(at):
            for t in (0, L):
                off = pl.ds(at + t, L)
                cs_v[off] = negv
                cidx_v[off] = bigv

        pad_live(cnt)
        ss_v[pl.ds(0, L)] = negv
        si_v[pl.ds(0, L)] = bigv

        def ib(v, carry):
            best_s, best_i, best_p = carry
            for u in range(2):
                base = 2 * v * L + u * L
                sl2 = pl.ds(base, L)
                best_s, best_i, best_p = arg_update(
                    cs_v[sl2], cidx_v[sl2], lane + base, best_s, best_i, best_p)
            return (best_s, best_i, best_p)

        nh0 = (cnt + 2 * L - 1) // (2 * L)
        mval0, bidx0, bpos0 = finalize(
            *lax.fori_loop(0, nh0, ib, (negv, bigv, zi)))

        def ocond(st):
            return st[0] > np.float32(-1e38)

        def obody(st):
            mval, bidx, bpos, m, ksel, nsup = st
            nh = (m + 2 * L - 1) // (2 * L)

            def icond(st2):
                return (st2[0] > np.float32(-1e38)) & (st2[5] < KCOMP)

            def ibody(st2):
                mval, bidx, bpos, ksel, nsup, k = st2
                pv = jnp.full((L,), bpos, jnp.int32)
                x1s = plsc.load_gather(cx1_v, [pv])
                y1s = plsc.load_gather(cy1_v, [pv])
                x2s = plsc.load_gather(cx2_v, [pv])
                y2s = plsc.load_gather(cy2_v, [pv])
                ars = plsc.load_gather(car_v, [pv])

                nsv = (nsup + L - 1) // L

                def rb(v, rkv):
                    sl2 = pl.ds(v * L, L)
                    ssv = ss_v[sl2]
                    siv = si_v[sl2]
                    before = (ssv > mval) | ((ssv == mval) & (siv < bidx))
                    return rkv + jnp.where(before, 1, 0)

                rank = ksel + jnp.sum(lax.fori_loop(0, nsv, rb, zi))

                rv = jnp.full((L,), rank, jnp.int32)
                zrow = jnp.zeros((L,), jnp.int32)
                plsc.store_scatter(o0_v, [zrow, rv], x1s)
                plsc.store_scatter(o1_v, [zrow, rv], y1s)
                plsc.store_scatter(o2_v, [zrow, rv], x2s)
                plsc.store_scatter(o3_v, [zrow, rv], y2s)
                plsc.store_scatter(o4_v, [zrow, rv],
                                   jnp.full((L,), mval, jnp.float32))

                def fb(v, carry):
                    best_s, best_i, best_p, nsc = carry
                    for u in range(2):
                        base = 2 * v * L + u * L
                        sl2 = pl.ds(base, L)
                        x1 = cx1_v[sl2]
                        y1 = cy1_v[sl2]
                        x2 = cx2_v[sl2]
                        y2 = cy2_v[sl2]
                        ar = car_v[sl2]
                        iv = cidx_v[sl2]
                        sv = cs_v[sl2]
                        ix1 = jnp.maximum(x1s, x1)
                        iy1 = jnp.maximum(y1s, y1)
                        ix2 = jnp.minimum(x2s, x2)
                        iy2 = jnp.minimum(y2s, y2)
                        inter = (jnp.maximum(ix2 - ix1, 0.0)
                                 * jnp.maximum(iy2 - iy1, 0.0))
                        union = jnp.maximum(ars + ar - inter, 1e-9)
                        hit = inter > TH_IOU * union
                        alive = sv > np.float32(-1e38)
                        issel = iv == bidx
                        ap = hit & alive & (~issel)
                        ci = lax.cumsum(ap.astype(jnp.int32))
                        spos = jnp.where(ap, nsc + ci - 1, DUMP)
                        plsc.store_scatter(ss_v, [spos], sv)
                        plsc.store_scatter(si_v, [spos], iv)
                        nsc = nsc + jnp.max(ci)
                        cs_v[sl2] = jnp.where(hit | issel, NEG, sv)
                        best_s, best_i, best_p = arg_update(
                            jnp.where(hit | issel, NEG, sv), iv, lane + base,
                            best_s, best_i, best_p)
                    return (best_s, best_i, best_p, nsc)

                best_s, best_i, best_p, nsup = lax.fori_loop(
                    0, nh, fb, (negv, bigv, zi, nsup))
                soff = pl.ds(nsup, L)
                ss_v[soff] = negv
                si_v[soff] = bigv

                nmval, nbidx, nbpos = finalize(best_s, best_i, best_p)
                return (nmval, nbidx, nbpos, ksel + 1, nsup, k + 1)

            st2 = lax.while_loop(
                icond, ibody, (mval, bidx, bpos, ksel, nsup, np.int32(0)))
            mval, bidx, bpos, ksel, nsup = st2[:5]

            def cb(v, carry):
                best_s, best_i, best_p, nc = carry
                for u in range(2):
                    base = 2 * v * L + u * L
                    sl2 = pl.ds(base, L)
                    sv = cs_v[sl2]
                    iv = cidx_v[sl2]
                    x1 = cx1_v[sl2]
                    y1 = cy1_v[sl2]
                    x2 = cx2_v[sl2]
                    y2 = cy2_v[sl2]
                    ar = car_v[sl2]
                    alive = sv > np.float32(-1e38)
                    ci = lax.cumsum(alive.astype(jnp.int32))
                    npos = jnp.where(alive, nc + ci - 1, DUMP)
                    plsc.store_scatter(cx1_v, [npos], x1)
                    plsc.store_scatter(cy1_v, [npos], y1)
                    plsc.store_scatter(cx2_v, [npos], x2)
                    plsc.store_scatter(cy2_v, [npos], y2)
                    plsc.store_scatter(car_v, [npos], ar)
                    plsc.store_scatter(cs_v, [npos], sv)
                    plsc.store_scatter(cidx_v, [npos], iv)
                    best_s, best_i, best_p = arg_update(
                        sv, iv, npos, best_s, best_i, best_p)
                    nc = nc + jnp.max(ci)
                return (best_s, best_i, best_p, nc)

            best_s, best_i, best_p, newm = lax.fori_loop(
                0, nh, cb, (negv, bigv, zi, np.int32(0)))
            pad_live(newm)
            mval, bidx, bpos = finalize(best_s, best_i, best_p)
            return (mval, bidx, bpos, newm, ksel, nsup)

        lax.while_loop(ocond, obody,
                       (mval0, bidx0, bpos0, cnt, np.int32(0), np.int32(0)))

        for j, o in enumerate((o0_v, o1_v, o2_v, o3_v, o4_v)):
            pltpu.sync_copy(o, out_hbm.at[pair, pl.ds(j, 1)])

    def pair_loop(t, _):
        pair = wid + t * NWORK

        @pl.when(pair < NPAIR)
        def _():
            do_pair(pair)
        return np.int32(0)

    lax.fori_loop(0, (NPAIR + NWORK - 1) // NWORK, pair_loop, np.int32(0))


@jax.jit
def kernel(conf, loc, anchors):
    padn = NPAD - NBOX
    pad_cls = jnp.where(jnp.arange(NC) == 0, 100.0, -100.0).astype(jnp.float32)
    conf_p = jnp.concatenate(
        [conf, jnp.broadcast_to(pad_cls, (NB, padn, NC))], axis=1)
    conf_t = jnp.transpose(conf_p, (0, 2, 1))
    loc_t = jnp.transpose(
        jnp.pad(loc, ((0, 0), (0, padn), (0, 0))), (0, 2, 1))
    anch_t = jnp.transpose(
        jnp.pad(anchors, ((0, padn), (0, 0))), (1, 0))

    mesh = plsc.VectorSubcoreMesh(core_axis_name="c", subcore_axis_name="s",
                                  num_cores=2, num_subcores=16)
    out = pl.kernel(
        _body,
        out_type=jax.ShapeDtypeStruct((NPAIR, 5, NPAD), jnp.float32),
        mesh=mesh,
        compiler_params=pltpu.CompilerParams(needs_layout_passes=False),
        scratch_types=[
            pltpu.VMEM((NC, NPAD), jnp.float32),
            pltpu.VMEM((1, NPAD), jnp.float32),
            pltpu.VMEM((4, NPAD), jnp.float32),
            pltpu.VMEM((4, NPAD), jnp.float32),
        ] + [pltpu.VMEM((PLEN,), jnp.float32)] * 7
          + [pltpu.VMEM((PLEN,), jnp.int32)] * 2
          + [pltpu.VMEM((1, NPAD), jnp.float32)] * 5,
    )(conf_t, loc_t, anch_t)

    return (out[:, :, :NBOX]
            .reshape(NB, NCLS, 5, NBOX)
            .transpose(0, 1, 3, 2))


# speedup vs baseline: 1.3131x; 1.3131x over previous
import functools

import numpy as np
import jax
import jax.numpy as jnp
from jax import lax
from jax.experimental import pallas as pl
from jax.experimental.pallas import tpu as pltpu
from jax.experimental.pallas import tpu_sc as plsc

NBOX = 1000
L = 16
NPAD = 1024
NCHUNK = NPAD // L
NB = 4
NC = 21
NCLS = NC - 1
NPAIR = NB * NCLS
NWORK = 32
TH_CONF = 0.05
TH_IOU = 0.5
NEG = float("-inf")
BIGI = np.int32(2**30)
PLEN = NPAD + 2 * L
DUMP = PLEN - 1
KCOMP = 24


def _body(conf_hbm, loc_hbm, anch_hbm, out_hbm,
          conf_v, cls_v, loc_v, anch_v,
          cx1_v, cy1_v, cx2_v, cy2_v, car_v, cs_v, cs0_v, cidx_v, cidx0_v,
          o0_v, o1_v, o2_v, o3_v, o4_v):
    cid = lax.axis_index("c")
    sid = lax.axis_index("s")
    wid = sid * 2 + cid
    lane = lax.iota(jnp.int32, L)

    pltpu.sync_copy(anch_hbm, anch_v)

    negv = jnp.full((L,), NEG, jnp.float32)
    bigv = jnp.full((L,), BIGI, jnp.int32)
    zi = jnp.zeros((L,), jnp.int32)

    def arg_update(sv, iv, posv, best_s, best_i, best_p):
        c2 = (sv > best_s) | ((sv == best_s) & (iv < best_i))
        return (jnp.where(c2, sv, best_s),
                jnp.where(c2, iv, best_i),
                jnp.where(c2, posv, best_p))

    def finalize(best_s, best_i, best_p):
        mval = jnp.max(best_s)
        cand = jnp.where(best_s == mval, best_i, BIGI)
        bidx = jnp.min(cand)
        bpos = jnp.min(jnp.where(cand == bidx, best_p, BIGI))
        return (mval, bidx, bpos)

    def do_pair(pair):
        b = pair // NCLS
        cls = pair % NCLS + 1
        pltpu.sync_copy(conf_hbm.at[b], conf_v)
        pltpu.sync_copy(conf_hbm.at[b, pl.ds(cls, 1)], cls_v)
        pltpu.sync_copy(loc_hbm.at[b], loc_v)

        def chunk_body(i, cnt):
            sl = pl.ds(i * L, L)
            m = conf_v[0, sl]
            for c in range(1, NC):
                m = jnp.maximum(m, conf_v[c, sl])
            z = jnp.exp(conf_v[0, sl] - m)
            for c in range(1, NC):
                z = z + jnp.exp(conf_v[c, sl] - m)
            s = jnp.exp(cls_v[0, sl] - m) / z

            a0 = anch_v[0, sl]
            a1 = anch_v[1, sl]
            a2 = anch_v[2, sl]
            a3 = anch_v[3, sl]
            cx = a0 + loc_v[0, sl] * 0.1 * a2
            cy = a1 + loc_v[1, sl] * 0.1 * a3
            w = a2 * jnp.exp(loc_v[2, sl] * 0.2)
            h = a3 * jnp.exp(loc_v[3, sl] * 0.2)
            x1 = cx - w / 2.0
            y1 = cy - h / 2.0
            x2 = cx + w / 2.0
            y2 = cy + h / 2.0
            area = (x2 - x1) * (y2 - y1)

            mask = s >= TH_CONF
            csum = lax.cumsum(mask.astype(jnp.int32))
            pos = jnp.where(mask, cnt + csum - 1, DUMP)
            plsc.store_scatter(cx1_v, [pos], x1)
            plsc.store_scatter(cy1_v, [pos], y1)
            plsc.store_scatter(cx2_v, [pos], x2)
            plsc.store_scatter(cy2_v, [pos], y2)
            plsc.store_scatter(car_v, [pos], area)
            plsc.store_scatter(cs_v, [pos], s)
            plsc.store_scatter(cs0_v, [pos], s)
            plsc.store_scatter(cidx_v, [pos], lane + i * L)
            plsc.store_scatter(cidx0_v, [pos], lane + i * L)

            zv = jnp.zeros((L,), jnp.float32)
            for o in (o0_v, o1_v, o2_v, o3_v, o4_v):
                o[0, sl] = zv

            return cnt + jnp.max(csum)

        cnt = lax.fori_loop(0, NCHUNK, chunk_body, np.int32(0))

        def pad_live(at):
            for t in (0, L):
                off = pl.ds(at + t, L)
                cs_v[off] = negv
                cidx_v[off] = bigv

        pad_live(cnt)
        for t in (0, L):
            off = pl.ds(cnt + t, L)
            cs0_v[off] = negv
            cidx0_v[off] = bigv

        def ib(v, carry):
            best_s, best_i, best_p = carry
            for u in range(2):
                base = 2 * v * L + u * L
                sl2 = pl.ds(base, L)
                best_s, best_i, best_p = arg_update(
                    cs_v[sl2], cidx_v[sl2], lane + base, best_s, best_i, best_p)
            return (best_s, best_i, best_p)

        nh0 = (cnt + 2 * L - 1) // (2 * L)
        mval0, bidx0, bpos0 = finalize(
            *lax.fori_loop(0, nh0, ib, (negv, bigv, zi)))

        def ocond(st):
            return st[0] > np.float32(-1e38)

        def obody(st):
            mval, bidx, bpos, m = st
            nh = (m + 2 * L - 1) // (2 * L)

            def icond(st2):
                return (st2[0] > np.float32(-1e38)) & (st2[3] < KCOMP)

            def ibody(st2):
                mval, bidx, bpos, k = st2
                pv = jnp.full((L,), bpos, jnp.int32)
                x1s = plsc.load_gather(cx1_v, [pv])
                y1s = plsc.load_gather(cy1_v, [pv])
                x2s = plsc.load_gather(cx2_v, [pv])
                y2s = plsc.load_gather(cy2_v, [pv])
                ars = plsc.load_gather(car_v, [pv])

                def rb(v, rkv):
                    for u in range(2):
                        sl2 = pl.ds(2 * v * L + u * L, L)
                        s0 = cs0_v[sl2]
                        i0 = cidx0_v[sl2]
                        before = (s0 > mval) | ((s0 == mval) & (i0 < bidx))
                        rkv = rkv + jnp.where(before, 1, 0)
                    return rkv

                rank = jnp.sum(lax.fori_loop(0, nh0, rb, zi))

                rv = jnp.full((L,), rank, jnp.int32)
                zrow = jnp.zeros((L,), jnp.int32)
                plsc.store_scatter(o0_v, [zrow, rv], x1s)
                plsc.store_scatter(o1_v, [zrow, rv], y1s)
                plsc.store_scatter(o2_v, [zrow, rv], x2s)
                plsc.store_scatter(o3_v, [zrow, rv], y2s)
                plsc.store_scatter(o4_v, [zrow, rv],
                                   jnp.full((L,), mval, jnp.float32))

                def fb(v, carry):
                    best_s, best_i, best_p = carry
                    for u in range(2):
                        base = 2 * v * L + u * L
                        sl2 = pl.ds(base, L)
                        x1 = cx1_v[sl2]
                        y1 = cy1_v[sl2]
                        x2 = cx2_v[sl2]
                        y2 = cy2_v[sl2]
                        ar = car_v[sl2]
                        iv = cidx_v[sl2]
                        sv = cs_v[sl2]
                        ix1 = jnp.maximum(x1s, x1)
                        iy1 = jnp.maximum(y1s, y1)
                        ix2 = jnp.minimum(x2s, x2)
                        iy2 = jnp.minimum(y2s, y2)
                        inter = (jnp.maximum(ix2 - ix1, 0.0)
                                 * jnp.maximum(iy2 - iy1, 0.0))
                        union = jnp.maximum(ars + ar - inter, 1e-9)
                        hit = inter > TH_IOU * union
                        sv2 = jnp.where(hit | (iv == bidx), NEG, sv)
                        cs_v[sl2] = sv2
                        best_s, best_i, best_p = arg_update(
                            sv2, iv, lane + base, best_s, best_i, best_p)
                    return (best_s, best_i, best_p)

                best_s, best_i, best_p = lax.fori_loop(
                    0, nh, fb, (negv, bigv, zi))

                nmval, nbidx, nbpos = finalize(best_s, best_i, best_p)
                return (nmval, nbidx, nbpos, k + 1)

            st2 = lax.while_loop(
                icond, ibody, (mval, bidx, bpos, np.int32(0)))
            mval, bidx, bpos = st2[:3]

            def cb(v, carry):
                best_s, best_i, best_p, nc = carry
                for u in range(2):
                    base = 2 * v * L + u * L
                    sl2 = pl.ds(base, L)
                    sv = cs_v[sl2]
                    iv = cidx_v[sl2]
                    x1 = cx1_v[sl2]
                    y1 = cy1_v[sl2]
                    x2 = cx2_v[sl2]
                    y2 = cy2_v[sl2]
                    ar = car_v[sl2]
                    alive = sv > np.float32(-1e38)
                    ci = lax.cumsum(alive.astype(jnp.int32))
                    npos = jnp.where(alive, nc + ci - 1, DUMP)
                    plsc.store_scatter(cx1_v, [npos], x1)
                    plsc.store_scatter(cy1_v, [npos], y1)
                    plsc.store_scatter(cx2_v, [npos], x2)
                    plsc.store_scatter(cy2_v, [npos], y2)
                    plsc.store_scatter(car_v, [npos], ar)
                    plsc.store_scatter(cs_v, [npos], sv)
                    plsc.store_scatter(cidx_v, [npos], iv)
                    best_s, best_i, best_p = arg_update(
                        sv, iv, npos, best_s, best_i, best_p)
                    nc = nc + jnp.max(ci)
                return (best_s, best_i, best_p, nc)

            best_s, best_i, best_p, newm = lax.fori_loop(
                0, nh, cb, (negv, bigv, zi, np.int32(0)))
            pad_live(newm)
            mval, bidx, bpos = finalize(best_s, best_i, best_p)
            return (mval, bidx, bpos, newm)

        lax.while_loop(ocond, obody, (mval0, bidx0, bpos0, cnt))

        for j, o in enumerate((o0_v, o1_v, o2_v, o3_v, o4_v)):
            pltpu.sync_copy(o, out_hbm.at[pair, pl.ds(j, 1)])

    def pair_loop(t, _):
        pair = wid + t * NWORK

        @pl.when(pair < NPAIR)
        def _():
            do_pair(pair)
        return np.int32(0)

    lax.fori_loop(0, (NPAIR + NWORK - 1) // NWORK, pair_loop, np.int32(0))


@jax.jit
def kernel(conf, loc, anchors):
    padn = NPAD - NBOX
    pad_cls = jnp.where(jnp.arange(NC) == 0, 100.0, -100.0).astype(jnp.float32)
    conf_p = jnp.concatenate(
        [conf, jnp.broadcast_to(pad_cls, (NB, padn, NC))], axis=1)
    conf_t = jnp.transpose(conf_p, (0, 2, 1))
    loc_t = jnp.transpose(
        jnp.pad(loc, ((0, 0), (0, padn), (0, 0))), (0, 2, 1))
    anch_t = jnp.transpose(
        jnp.pad(anchors, ((0, padn), (0, 0))), (1, 0))

    mesh = plsc.VectorSubcoreMesh(core_axis_name="c", subcore_axis_name="s",
                                  num_cores=2, num_subcores=16)
    out = pl.kernel(
        _body,
        out_type=jax.ShapeDtypeStruct((NPAIR, 5, NPAD), jnp.float32),
        mesh=mesh,
        compiler_params=pltpu.CompilerParams(needs_layout_passes=False),
        scratch_types=[
            pltpu.VMEM((NC, NPAD), jnp.float32),
            pltpu.VMEM((1, NPAD), jnp.float32),
            pltpu.VMEM((4, NPAD), jnp.float32),
            pltpu.VMEM((4, NPAD), jnp.float32),
        ] + [pltpu.VMEM((PLEN,), jnp.float32)] * 7
          + [pltpu.VMEM((PLEN,), jnp.int32)] * 2
          + [pltpu.VMEM((1, NPAD), jnp.float32)] * 5,
    )(conf_t, loc_t, anch_t)

    return (out[:, :, :NBOX]
            .reshape(NB, NCLS, 5, NBOX)
            .transpose(0, 1, 3, 2))


# speedup vs baseline: 1.3819x; 1.0524x over previous
import functools

import numpy as np
import jax
import jax.numpy as jnp
from jax import lax
from jax.experimental import pallas as pl
from jax.experimental.pallas import tpu as pltpu
from jax.experimental.pallas import tpu_sc as plsc

NBOX = 1000
L = 16
NPAD = 1024
NCHUNK = NPAD // L
NB = 4
NC = 21
NCLS = NC - 1
NPAIR = NB * NCLS
NWORK = 32
TH_CONF = 0.05
TH_IOU = 0.5
NEG = float("-inf")
BIGI = np.int32(2**30)
PLEN = NPAD + 2 * L
DUMP = PLEN - 1
KCOMP = 24


def _body(conf_hbm, loc_hbm, anch_hbm, out_hbm,
          conf_v, cls_v, loc_v, anch_v,
          cx1_v, cy1_v, cx2_v, cy2_v, car_v, cs_v, cs0_v, cidx_v, cidx0_v,
          o0_v, o1_v, o2_v, o3_v, o4_v):
    cid = lax.axis_index("c")
    sid = lax.axis_index("s")
    wid = sid * 2 + cid
    lane = lax.iota(jnp.int32, L)

    pltpu.sync_copy(anch_hbm, anch_v)

    negv = jnp.full((L,), NEG, jnp.float32)
    bigv = jnp.full((L,), BIGI, jnp.int32)
    zi = jnp.zeros((L,), jnp.int32)

    def arg_update(sv, posv, best_s, best_p):
        c2 = sv > best_s
        return (jnp.where(c2, sv, best_s), jnp.where(c2, posv, best_p))

    def finalize(best_s, best_p):
        mval = jnp.max(best_s)
        bpos = jnp.min(jnp.where(best_s == mval, best_p, BIGI))
        return (mval, bpos)

    def do_pair(pair):
        b = pair // NCLS
        cls = pair % NCLS + 1
        pltpu.sync_copy(conf_hbm.at[b], conf_v)
        pltpu.sync_copy(conf_hbm.at[b, pl.ds(cls, 1)], cls_v)
        pltpu.sync_copy(loc_hbm.at[b], loc_v)

        def chunk_body(i, cnt):
            sl = pl.ds(i * L, L)
            m = conf_v[0, sl]
            for c in range(1, NC):
                m = jnp.maximum(m, conf_v[c, sl])
            z = jnp.exp(conf_v[0, sl] - m)
            for c in range(1, NC):
                z = z + jnp.exp(conf_v[c, sl] - m)
            s = jnp.exp(cls_v[0, sl] - m) / z

            a0 = anch_v[0, sl]
            a1 = anch_v[1, sl]
            a2 = anch_v[2, sl]
            a3 = anch_v[3, sl]
            cx = a0 + loc_v[0, sl] * 0.1 * a2
            cy = a1 + loc_v[1, sl] * 0.1 * a3
            w = a2 * jnp.exp(loc_v[2, sl] * 0.2)
            h = a3 * jnp.exp(loc_v[3, sl] * 0.2)
            x1 = cx - w / 2.0
            y1 = cy - h / 2.0
            x2 = cx + w / 2.0
            y2 = cy + h / 2.0
            area = (x2 - x1) * (y2 - y1)

            mask = s >= TH_CONF
            csum = lax.cumsum(mask.astype(jnp.int32))
            pos = jnp.where(mask, cnt + csum - 1, DUMP)
            plsc.store_scatter(cx1_v, [pos], x1)
            plsc.store_scatter(cy1_v, [pos], y1)
            plsc.store_scatter(cx2_v, [pos], x2)
            plsc.store_scatter(cy2_v, [pos], y2)
            plsc.store_scatter(car_v, [pos], area)
            plsc.store_scatter(cs_v, [pos], s)
            plsc.store_scatter(cs0_v, [pos], s)
            plsc.store_scatter(cidx_v, [pos], lane + i * L)
            plsc.store_scatter(cidx0_v, [pos], lane + i * L)

            zv = jnp.zeros((L,), jnp.float32)
            for o in (o0_v, o1_v, o2_v, o3_v, o4_v):
                o[0, sl] = zv

            return cnt + jnp.max(csum)

        cnt = lax.fori_loop(0, NCHUNK, chunk_body, np.int32(0))

        def pad_live(at):
            for t in (0, L):
                off = pl.ds(at + t, L)
                cs_v[off] = negv
                cidx_v[off] = bigv

        pad_live(cnt)
        for t in (0, L):
            off = pl.ds(cnt + t, L)
            cs0_v[off] = negv
            cidx0_v[off] = bigv

        def ib(v, carry):
            best_s, best_p = carry
            for u in range(2):
                base = 2 * v * L + u * L
                best_s, best_p = arg_update(
                    cs_v[pl.ds(base, L)], lane + base, best_s, best_p)
            return (best_s, best_p)

        nh0 = (cnt + 2 * L - 1) // (2 * L)
        mval0, bpos0 = finalize(*lax.fori_loop(0, nh0, ib, (negv, zi)))

        def ocond(st):
            return st[0] > np.float32(-1e38)

        def obody(st):
            mval, bpos, m = st
            nh = (m + 2 * L - 1) // (2 * L)

            def icond(st2):
                return (st2[0] > np.float32(-1e38)) & (st2[2] < KCOMP)

            def ibody(st2):
                mval, bpos, k = st2
                pv = jnp.full((L,), bpos, jnp.int32)
                x1s = plsc.load_gather(cx1_v, [pv])
                y1s = plsc.load_gather(cy1_v, [pv])
                x2s = plsc.load_gather(cx2_v, [pv])
                y2s = plsc.load_gather(cy2_v, [pv])
                ars = plsc.load_gather(car_v, [pv])
                bidx = plsc.load_gather(cidx_v, [pv])

                def rb(v, rkv):
                    for u in range(2):
                        sl2 = pl.ds(2 * v * L + u * L, L)
                        s0 = cs0_v[sl2]
                        i0 = cidx0_v[sl2]
                        before = (s0 > mval) | ((s0 == mval) & (i0 < bidx))
                        rkv = rkv + jnp.where(before, 1, 0)
                    return rkv

                rank = jnp.sum(lax.fori_loop(0, nh0, rb, zi))

                rv = jnp.full((L,), rank, jnp.int32)
                zrow = jnp.zeros((L,), jnp.int32)
                plsc.store_scatter(o0_v, [zrow, rv], x1s)
                plsc.store_scatter(o1_v, [zrow, rv], y1s)
                plsc.store_scatter(o2_v, [zrow, rv], x2s)
                plsc.store_scatter(o3_v, [zrow, rv], y2s)
                plsc.store_scatter(o4_v, [zrow, rv],
                                   jnp.full((L,), mval, jnp.float32))

                def fb(v, carry):
                    best_s, best_p = carry
                    for u in range(2):
                        base = 2 * v * L + u * L
                        sl2 = pl.ds(base, L)
                        x1 = cx1_v[sl2]
                        y1 = cy1_v[sl2]
                        x2 = cx2_v[sl2]
                        y2 = cy2_v[sl2]
                        ar = car_v[sl2]
                        sv = cs_v[sl2]
                        ix1 = jnp.maximum(x1s, x1)
                        iy1 = jnp.maximum(y1s, y1)
                        ix2 = jnp.minimum(x2s, x2)
                        iy2 = jnp.minimum(y2s, y2)
                        inter = (jnp.maximum(ix2 - ix1, 0.0)
                                 * jnp.maximum(iy2 - iy1, 0.0))
                        union = jnp.maximum(ars + ar - inter, 1e-9)
                        hit = inter > TH_IOU * union
                        posv = lane + base
                        sv2 = jnp.where(hit | (posv == bpos), NEG, sv)
                        cs_v[sl2] = sv2
                        best_s, best_p = arg_update(sv2, posv, best_s, best_p)
                    return (best_s, best_p)

                best_s, best_p = lax.fori_loop(0, nh, fb, (negv, zi))

                nmval, nbpos = finalize(best_s, best_p)
                return (nmval, nbpos, k + 1)

            st2 = lax.while_loop(icond, ibody, (mval, bpos, np.int32(0)))
            mval, bpos = st2[:2]

            def cb(v, carry):
                best_s, best_p, nc = carry
                for u in range(2):
                    base = 2 * v * L + u * L
                    sl2 = pl.ds(base, L)
                    sv = cs_v[sl2]
                    iv = cidx_v[sl2]
                    x1 = cx1_v[sl2]
                    y1 = cy1_v[sl2]
                    x2 = cx2_v[sl2]
                    y2 = cy2_v[sl2]
                    ar = car_v[sl2]
                    alive = sv > np.float32(-1e38)
                    ci = lax.cumsum(alive.astype(jnp.int32))
                    npos = jnp.where(alive, nc + ci - 1, DUMP)
                    plsc.store_scatter(cx1_v, [npos], x1)
                    plsc.store_scatter(cy1_v, [npos], y1)
                    plsc.store_scatter(cx2_v, [npos], x2)
                    plsc.store_scatter(cy2_v, [npos], y2)
                    plsc.store_scatter(car_v, [npos], ar)
                    plsc.store_scatter(cs_v, [npos], sv)
                    plsc.store_scatter(cidx_v, [npos], iv)
                    best_s, best_p = arg_update(sv, npos, best_s, best_p)
                    nc = nc + jnp.max(ci)
                return (best_s, best_p, nc)

            best_s, best_p, newm = lax.fori_loop(
                0, nh, cb, (negv, zi, np.int32(0)))
            pad_live(newm)
            mval, bpos = finalize(best_s, best_p)
            return (mval, bpos, newm)

        lax.while_loop(ocond, obody, (mval0, bpos0, cnt))

        for j, o in enumerate((o0_v, o1_v, o2_v, o3_v, o4_v)):
            pltpu.sync_copy(o, out_hbm.at[pair, pl.ds(j, 1)])

    def pair_loop(t, _):
        pair = wid + t * NWORK

        @pl.when(pair < NPAIR)
        def _():
            do_pair(pair)
        return np.int32(0)

    lax.fori_loop(0, (NPAIR + NWORK - 1) // NWORK, pair_loop, np.int32(0))


@jax.jit
def kernel(conf, loc, anchors):
    padn = NPAD - NBOX
    pad_cls = jnp.where(jnp.arange(NC) == 0, 100.0, -100.0).astype(jnp.float32)
    conf_p = jnp.concatenate(
        [conf, jnp.broadcast_to(pad_cls, (NB, padn, NC))], axis=1)
    conf_t = jnp.transpose(conf_p, (0, 2, 1))
    loc_t = jnp.transpose(
        jnp.pad(loc, ((0, 0), (0, padn), (0, 0))), (0, 2, 1))
    anch_t = jnp.transpose(
        jnp.pad(anchors, ((0, padn), (0, 0))), (1, 0))

    mesh = plsc.VectorSubcoreMesh(core_axis_name="c", subcore_axis_name="s",
                                  num_cores=2, num_subcores=16)
    out = pl.kernel(
        _body,
        out_type=jax.ShapeDtypeStruct((NPAIR, 5, NPAD), jnp.float32),
        mesh=mesh,
        compiler_params=pltpu.CompilerParams(needs_layout_passes=False),
        scratch_types=[
            pltpu.VMEM((NC, NPAD), jnp.float32),
            pltpu.VMEM((1, NPAD), jnp.float32),
            pltpu.VMEM((4, NPAD), jnp.float32),
            pltpu.VMEM((4, NPAD), jnp.float32),
        ] + [pltpu.VMEM((PLEN,), jnp.float32)] * 7
          + [pltpu.VMEM((PLEN,), jnp.int32)] * 2
          + [pltpu.VMEM((1, NPAD), jnp.float32)] * 5,
    )(conf_t, loc_t, anch_t)

    return (out[:, :, :NBOX]
            .reshape(NB, NCLS, 5, NBOX)
            .transpose(0, 1, 3, 2))


# speedup vs baseline: 2.1006x; 1.5201x over previous
import functools

import numpy as np
import jax
import jax.numpy as jnp
from jax import lax
from jax.experimental import pallas as pl
from jax.experimental.pallas import tpu as pltpu
from jax.experimental.pallas import tpu_sc as plsc

NBOX = 1000
L = 16
NPAD = 1024
NCHUNK = NPAD // L
NB = 4
NC = 21
NCLS = NC - 1
NPAIR = NB * NCLS
NWORK = 32
TH_CONF = 0.05
TH_IOU = 0.5
NEG = float("-inf")
BIGI = np.int32(2**30)
UNR = 4
PLEN = NPAD + UNR * L
DUMP = PLEN - 1
KCOMP = 32


def _body(conf_hbm, loc_hbm, anch_hbm, out_hbm,
          conf_v, cls_v, loc_v, anch_v,
          cx1_v, cy1_v, cx2_v, cy2_v, car_v, cs_v, cs0_v, cidx_v, cidx0_v,
          o0_v, o1_v, o2_v, o3_v, o4_v, wq_s):
    cid = lax.axis_index("c")
    sid = lax.axis_index("s")
    wid = sid * 2 + cid
    lane = lax.iota(jnp.int32, L)

    pltpu.sync_copy(anch_hbm, anch_v)

    negv = jnp.full((L,), NEG, jnp.float32)
    bigv = jnp.full((L,), BIGI, jnp.int32)
    zi = jnp.zeros((L,), jnp.int32)

    def arg_update(sv, posv, best_s, best_p):
        c2 = sv > best_s
        return (jnp.where(c2, sv, best_s), jnp.where(c2, posv, best_p))

    def finalize(best_s, best_p):
        mval = jnp.max(best_s)
        bpos = jnp.min(jnp.where(best_s == mval, best_p, BIGI))
        return (mval, bpos)

    def do_pair(pair):
        b = pair // NCLS
        cls = pair % NCLS + 1
        pltpu.sync_copy(conf_hbm.at[b], conf_v)
        pltpu.sync_copy(conf_hbm.at[b, pl.ds(cls, 1)], cls_v)
        pltpu.sync_copy(loc_hbm.at[b], loc_v)

        def chunk_body(i, cnt):
            sl = pl.ds(i * L, L)
            m = conf_v[0, sl]
            for c in range(1, NC):
                m = jnp.maximum(m, conf_v[c, sl])
            z = jnp.exp(conf_v[0, sl] - m)
            for c in range(1, NC):
                z = z + jnp.exp(conf_v[c, sl] - m)
            s = jnp.exp(cls_v[0, sl] - m) / z

            a0 = anch_v[0, sl]
            a1 = anch_v[1, sl]
            a2 = anch_v[2, sl]
            a3 = anch_v[3, sl]
            cx = a0 + loc_v[0, sl] * 0.1 * a2
            cy = a1 + loc_v[1, sl] * 0.1 * a3
            w = a2 * jnp.exp(loc_v[2, sl] * 0.2)
            h = a3 * jnp.exp(loc_v[3, sl] * 0.2)
            x1 = cx - w / 2.0
            y1 = cy - h / 2.0
            x2 = cx + w / 2.0
            y2 = cy + h / 2.0
            area = (x2 - x1) * (y2 - y1)

            mask = s >= TH_CONF
            csum = lax.cumsum(mask.astype(jnp.int32))
            pos = jnp.where(mask, cnt + csum - 1, DUMP)
            plsc.store_scatter(cx1_v, [pos], x1)
            plsc.store_scatter(cy1_v, [pos], y1)
            plsc.store_scatter(cx2_v, [pos], x2)
            plsc.store_scatter(cy2_v, [pos], y2)
            plsc.store_scatter(car_v, [pos], area)
            plsc.store_scatter(cs_v, [pos], s)
            plsc.store_scatter(cs0_v, [pos], s)
            plsc.store_scatter(cidx_v, [pos], lane + i * L)
            plsc.store_scatter(cidx0_v, [pos], lane + i * L)

            zv = jnp.zeros((L,), jnp.float32)
            for o in (o0_v, o1_v, o2_v, o3_v, o4_v):
                o[0, sl] = zv

            return cnt + jnp.max(csum)

        cnt = lax.fori_loop(0, NCHUNK, chunk_body, np.int32(0))

        def pad_live(at):
            for t in range(UNR):
                off = pl.ds(at + t * L, L)
                cs_v[off] = negv
                cidx_v[off] = bigv

        pad_live(cnt)
        for t in range(UNR):
            off = pl.ds(cnt + t * L, L)
            cs0_v[off] = negv
            cidx0_v[off] = bigv

        def ib(v, carry):
            best_s, best_p = carry
            for u in range(UNR):
                base = UNR * v * L + u * L
                best_s, best_p = arg_update(
                    cs_v[pl.ds(base, L)], lane + base, best_s, best_p)
            return (best_s, best_p)

        nh0 = (cnt + UNR * L - 1) // (UNR * L)
        mval0, bpos0 = finalize(*lax.fori_loop(0, nh0, ib, (negv, zi)))

        def ocond(st):
            return st[0] > np.float32(-1e38)

        def obody(st):
            mval, bpos, m = st
            nh = (m + UNR * L - 1) // (UNR * L)

            def icond(st2):
                return (st2[0] > np.float32(-1e38)) & (st2[2] < KCOMP)

            def ibody(st2):
                mval, bpos, k = st2
                pv = jnp.full((L,), bpos, jnp.int32)
                x1s = plsc.load_gather(cx1_v, [pv])
                y1s = plsc.load_gather(cy1_v, [pv])
                x2s = plsc.load_gather(cx2_v, [pv])
                y2s = plsc.load_gather(cy2_v, [pv])
                ars = plsc.load_gather(car_v, [pv])
                bidx = plsc.load_gather(cidx_v, [pv])

                def rb(v, rkv):
                    for u in range(UNR):
                        sl2 = pl.ds(UNR * v * L + u * L, L)
                        s0 = cs0_v[sl2]
                        i0 = cidx0_v[sl2]
                        before = (s0 > mval) | ((s0 == mval) & (i0 < bidx))
                        rkv = rkv + jnp.where(before, 1, 0)
                    return rkv

                rank = jnp.sum(lax.fori_loop(0, nh0, rb, zi))

                rv = jnp.full((L,), rank, jnp.int32)
                zrow = jnp.zeros((L,), jnp.int32)
                plsc.store_scatter(o0_v, [zrow, rv], x1s)
                plsc.store_scatter(o1_v, [zrow, rv], y1s)
                plsc.store_scatter(o2_v, [zrow, rv], x2s)
                plsc.store_scatter(o3_v, [zrow, rv], y2s)
                plsc.store_scatter(o4_v, [zrow, rv],
                                   jnp.full((L,), mval, jnp.float32))

                def fb(v, carry):
                    best_s, best_p = carry
                    for u in range(UNR):
                        base = UNR * v * L + u * L
                        sl2 = pl.ds(base, L)
                        x1 = cx1_v[sl2]
                        y1 = cy1_v[sl2]
                        x2 = cx2_v[sl2]
                        y2 = cy2_v[sl2]
                        ar = car_v[sl2]
                        sv = cs_v[sl2]
                        ix1 = jnp.maximum(x1s, x1)
                        iy1 = jnp.maximum(y1s, y1)
                        ix2 = jnp.minimum(x2s, x2)
                        iy2 = jnp.minimum(y2s, y2)
                        inter = (jnp.maximum(ix2 - ix1, 0.0)
                                 * jnp.maximum(iy2 - iy1, 0.0))
                        union = jnp.maximum(ars + ar - inter, 1e-9)
                        hit = inter > TH_IOU * union
                        posv = lane + base
                        sv2 = jnp.where(hit | (posv == bpos), NEG, sv)
                        cs_v[sl2] = sv2
                        best_s, best_p = arg_update(sv2, posv, best_s, best_p)
                    return (best_s, best_p)

                best_s, best_p = lax.fori_loop(0, nh, fb, (negv, zi))

                nmval, nbpos = finalize(best_s, best_p)
                return (nmval, nbpos, k + 1)

            st2 = lax.while_loop(icond, ibody, (mval, bpos, np.int32(0)))
            mval, bpos = st2[:2]

            def cb(v, carry):
                best_s, best_p, nc = carry
                for u in range(UNR):
                    base = UNR * v * L + u * L
                    sl2 = pl.ds(base, L)
                    sv = cs_v[sl2]
                    iv = cidx_v[sl2]
                    x1 = cx1_v[sl2]
                    y1 = cy1_v[sl2]
                    x2 = cx2_v[sl2]
                    y2 = cy2_v[sl2]
                    ar = car_v[sl2]
                    alive = sv > np.float32(-1e38)
                    ci = lax.cumsum(alive.astype(jnp.int32))
                    npos = jnp.where(alive, nc + ci - 1, DUMP)
                    plsc.store_scatter(cx1_v, [npos], x1)
                    plsc.store_scatter(cy1_v, [npos], y1)
                    plsc.store_scatter(cx2_v, [npos], x2)
                    plsc.store_scatter(cy2_v, [npos], y2)
                    plsc.store_scatter(car_v, [npos], ar)
                    plsc.store_scatter(cs_v, [npos], sv)
                    plsc.store_scatter(cidx_v, [npos], iv)
                    best_s, best_p = arg_update(sv, npos, best_s, best_p)
                    nc = nc + jnp.max(ci)
                return (best_s, best_p, nc)

            best_s, best_p, newm = lax.fori_loop(
                0, nh, cb, (negv, zi, np.int32(0)))
            pad_live(newm)
            mval, bpos = finalize(best_s, best_p)
            return (mval, bpos, newm)

        lax.while_loop(ocond, obody, (mval0, bpos0, cnt))

        for j, o in enumerate((o0_v, o1_v, o2_v, o3_v, o4_v)):
            pltpu.sync_copy(o, out_hbm.at[pair, pl.ds(j, 1)])

    npc = NPAIR // 2

    @pl.when(sid == 0)
    def _():
        wq_s[0] = np.int32(0)
    plsc.subcore_barrier()

    def wcond(t):
        return t < npc

    def wbody(t):
        do_pair(cid * npc + t)
        return plsc.fetch_and_add(wq_s, np.int32(1), subcore_id=0)

    lax.while_loop(wcond, wbody, plsc.fetch_and_add(wq_s, np.int32(1), subcore_id=0))


@jax.jit
def kernel(conf, loc, anchors):
    padn = NPAD - NBOX
    pad_cls = jnp.where(jnp.arange(NC) == 0, 100.0, -100.0).astype(jnp.float32)
    conf_p = jnp.concatenate(
        [conf, jnp.broadcast_to(pad_cls, (NB, padn, NC))], axis=1)
    conf_t = jnp.transpose(conf_p, (0, 2, 1))
    loc_t = jnp.transpose(
        jnp.pad(loc, ((0, 0), (0, padn), (0, 0))), (0, 2, 1))
    anch_t = jnp.transpose(
        jnp.pad(anchors, ((0, padn), (0, 0))), (1, 0))

    mesh = plsc.VectorSubcoreMesh(core_axis_name="c", subcore_axis_name="s",
                                  num_cores=2, num_subcores=16)
    out = pl.kernel(
        _body,
        out_type=jax.ShapeDtypeStruct((NPAIR, 5, NPAD), jnp.float32),
        mesh=mesh,
        compiler_params=pltpu.CompilerParams(needs_layout_passes=False),
        scratch_types=[
            pltpu.VMEM((NC, NPAD), jnp.float32),
            pltpu.VMEM((1, NPAD), jnp.float32),
            pltpu.VMEM((4, NPAD), jnp.float32),
            pltpu.VMEM((4, NPAD), jnp.float32),
        ] + [pltpu.VMEM((PLEN,), jnp.float32)] * 7
          + [pltpu.VMEM((PLEN,), jnp.int32)] * 2
          + [pltpu.VMEM((1, NPAD), jnp.float32)] * 5
          + [pltpu.SMEM((1,), jnp.int32)],
    )(conf_t, loc_t, anch_t)

    return (out[:, :, :NBOX]
            .reshape(NB, NCLS, 5, NBOX)
            .transpose(0, 1, 3, 2))
